# baseline (speedup 1.0000x reference)
import functools

import jax
import jax.numpy as jnp
from jax import lax
from jax.experimental import pallas as pl
from jax.experimental.pallas import tpu as pltpu
from jax.experimental.pallas import tpu_sc as plsc

EMBED = 128
BATCH = 16384
CHUNK = 128
NC, NS = 2, 16
NW = NC * NS
NB = 4
NSEG = 2
BSEG = BATCH // NSEG

_sc_mesh = plsc.VectorSubcoreMesh(core_axis_name="c", subcore_axis_name="s")


def _gather_body(g_per_w, users_hbm, movies_hbm, uemb_hbm, memb_hbm,
                 u_out, m_out, idx_u, idx_m, rows, *sems):
    gsems, wsems = sems[:NB], sems[NB:]
    wid = lax.axis_index("s") * NC + lax.axis_index("c")
    gbase = wid * g_per_w
    pltpu.sync_copy(users_hbm.at[pl.ds(gbase, g_per_w)], idx_u)
    pltpu.sync_copy(movies_hbm.at[pl.ds(gbase, g_per_w)], idx_m)
    chunks = ([(idx_u, uemb_hbm, u_out, g) for g in range(g_per_w)]
              + [(idx_m, memb_hbm, m_out, g) for g in range(g_per_w)])
    n = len(chunks)
    gdesc, wdesc = [None] * n, [None] * n

    def issue_write(j):
        _, _, out, gj = chunks[j]
        bj = j % NB
        gdesc[j].wait()
        wdesc[j] = pltpu.async_copy(
            rows.at[bj], out.at[pl.ds((gbase + gj) * CHUNK, CHUNK)], wsems[bj])

    for c in range(n):
        b = c % NB
        if c >= NB:
            wdesc[c - NB].wait()
        ix, tab, _, g = chunks[c]
        gdesc[c] = pltpu.async_copy(tab.at[ix.at[g]], rows.at[b], gsems[b])
        if c - (NB - 1) >= 0:
            issue_write(c - (NB - 1))
    for j in range(max(n - (NB - 1), 0), n):
        issue_write(j)
    for j in range(max(n - NB, 0), n):
        wdesc[j].wait()


def _make_gather(batch):
    groups = batch // CHUNK
    g_per_w = groups // NW
    return pl.kernel(
        functools.partial(_gather_body, g_per_w),
        out_type=(
            jax.ShapeDtypeStruct((batch, EMBED), jnp.float32),
            jax.ShapeDtypeStruct((batch, EMBED), jnp.float32),
        ),
        mesh=_sc_mesh,
        scratch_types=(
            [pltpu.VMEM((g_per_w, CHUNK), jnp.int32),
             pltpu.VMEM((g_per_w, CHUNK), jnp.int32),
             pltpu.VMEM((NB, CHUNK, EMBED), jnp.float32)]
            + [pltpu.SemaphoreType.DMA] * (2 * NB)
        ),
    )


def _mm_body(u_ref, m_ref, wu_ref, wm_ref, b_ref, o_ref):
    dn = (((1,), (1,)), ((), ()))
    acc = lax.dot_general(wu_ref[...], u_ref[...], dn,
                          preferred_element_type=jnp.float32)
    acc = acc + lax.dot_general(wm_ref[...], m_ref[...], dn,
                                preferred_element_type=jnp.float32)
    o_ref[...] = acc + b_ref[...]


BM = 2048


def _matmul(u_rows, m_rows, wu, wm, b2):
    batch = u_rows.shape[0]
    n_out = wu.shape[0]
    return pl.pallas_call(
        _mm_body,
        grid=(batch // BM,),
        in_specs=[
            pl.BlockSpec((BM, EMBED), lambda i: (i, 0)),
            pl.BlockSpec((BM, EMBED), lambda i: (i, 0)),
            pl.BlockSpec((n_out, EMBED), lambda i: (0, 0)),
            pl.BlockSpec((n_out, EMBED), lambda i: (0, 0)),
            pl.BlockSpec((n_out, 1), lambda i: (0, 0)),
        ],
        out_specs=pl.BlockSpec((n_out, BM), lambda i: (0, i)),
        out_shape=jax.ShapeDtypeStruct((n_out, batch), jnp.float32),
    )(u_rows, m_rows, wu, wm, b2)


_gather_seg = _make_gather(BSEG)


def kernel(users, movies, user_emb, movie_emb, fc_w, fc_b):
    wu = fc_w[:, :EMBED]
    wm = fc_w[:, EMBED:]
    b2 = fc_b.reshape(-1, 1)
    groups_seg = BSEG // CHUNK
    outs = []
    for s in range(NSEG):
        us = lax.slice(users, (s * BSEG,), ((s + 1) * BSEG,))
        ms = lax.slice(movies, (s * BSEG,), ((s + 1) * BSEG,))
        u_rows, m_rows = _gather_seg(us.reshape(groups_seg, CHUNK),
                                     ms.reshape(groups_seg, CHUNK),
                                     user_emb, movie_emb)
        outs.append(_matmul(u_rows, m_rows, wu, wm, b2))
    out_t = outs[0] if NSEG == 1 else jnp.concatenate(outs, axis=1)
    return out_t.T


# speedup vs baseline: 1.0930x; 1.0930x over previous
import functools

import jax
import jax.numpy as jnp
from jax import lax
from jax.experimental import pallas as pl
from jax.experimental.pallas import tpu as pltpu
from jax.experimental.pallas import tpu_sc as plsc

EMBED = 128
BATCH = 16384
CHUNK = 128
NC, NS = 2, 16
NW = NC * NS
NB = 3
NSEG = 1
BSEG = BATCH // NSEG

_sc_mesh = plsc.VectorSubcoreMesh(core_axis_name="c", subcore_axis_name="s")


def _gather_body(g_per_w, users_hbm, movies_hbm, uemb_hbm, memb_hbm,
                 u_out, m_out, idx_u, idx_m, rows, isem_u, isem_m, *sems):
    gsems, wsems = sems[:NB], sems[NB:]
    wid = lax.axis_index("s") * NC + lax.axis_index("c")
    gbase = wid * g_per_w
    icp_u = pltpu.async_copy(users_hbm.at[pl.ds(gbase, g_per_w)], idx_u, isem_u)
    icp_m = pltpu.async_copy(movies_hbm.at[pl.ds(gbase, g_per_w)], idx_m, isem_m)
    icp = {id(idx_u): icp_u, id(idx_m): icp_m}
    pairs = ([(idx_u, uemb_hbm, u_out, q) for q in range(g_per_w // 2)]
             + [(idx_m, memb_hbm, m_out, q) for q in range(g_per_w // 2)])
    n = len(pairs)
    gdesc, wdesc = [None] * n, [None] * n

    def issue_write(j):
        _, _, out, qj = pairs[j]
        bj = j % NB
        for d in gdesc[j]:
            d.wait()
        wdesc[j] = pltpu.async_copy(
            rows.at[bj],
            out.at[pl.ds((gbase // 2 + qj) * 2 * CHUNK, 2 * CHUNK)],
            wsems[bj])

    for c in range(n):
        b = c % NB
        if c >= NB:
            wdesc[c - NB].wait()
        ix, tab, _, q = pairs[c]
        d = icp.pop(id(ix), None)
        if d is not None:
            d.wait()
        gdesc[c] = [
            pltpu.async_copy(tab.at[ix.at[2 * q]],
                             rows.at[b, pl.ds(0, CHUNK)], gsems[b]),
            pltpu.async_copy(tab.at[ix.at[2 * q + 1]],
                             rows.at[b, pl.ds(CHUNK, CHUNK)], gsems[b]),
        ]
        if c - (NB - 1) >= 0:
            issue_write(c - (NB - 1))
    for j in range(max(n - (NB - 1), 0), n):
        issue_write(j)
    for j in range(max(n - NB, 0), n):
        wdesc[j].wait()


def _make_gather(batch):
    groups = batch // CHUNK
    g_per_w = groups // NW
    return pl.kernel(
        functools.partial(_gather_body, g_per_w),
        out_type=(
            jax.ShapeDtypeStruct((batch, EMBED), jnp.float32),
            jax.ShapeDtypeStruct((batch, EMBED), jnp.float32),
        ),
        mesh=_sc_mesh,
        scratch_types=(
            [pltpu.VMEM((g_per_w, CHUNK), jnp.int32),
             pltpu.VMEM((g_per_w, CHUNK), jnp.int32),
             pltpu.VMEM((NB, 2 * CHUNK, EMBED), jnp.float32),
             pltpu.SemaphoreType.DMA,
             pltpu.SemaphoreType.DMA]
            + [pltpu.SemaphoreType.DMA] * (2 * NB)
        ),
    )


def _mm_body(u_ref, m_ref, wu_ref, wm_ref, b_ref, o_ref):
    dn = (((1,), (1,)), ((), ()))
    acc = lax.dot_general(wu_ref[...], u_ref[...], dn,
                          preferred_element_type=jnp.float32)
    acc = acc + lax.dot_general(wm_ref[...], m_ref[...], dn,
                                preferred_element_type=jnp.float32)
    o_ref[...] = acc + b_ref[...]


BM = 2048


def _matmul(u_rows, m_rows, wu, wm, b2):
    batch = u_rows.shape[0]
    n_out = wu.shape[0]
    return pl.pallas_call(
        _mm_body,
        grid=(batch // BM,),
        in_specs=[
            pl.BlockSpec((BM, EMBED), lambda i: (i, 0)),
            pl.BlockSpec((BM, EMBED), lambda i: (i, 0)),
            pl.BlockSpec((n_out, EMBED), lambda i: (0, 0)),
            pl.BlockSpec((n_out, EMBED), lambda i: (0, 0)),
            pl.BlockSpec((n_out, 1), lambda i: (0, 0)),
        ],
        out_specs=pl.BlockSpec((n_out, BM), lambda i: (0, i)),
        out_shape=jax.ShapeDtypeStruct((n_out, batch), jnp.float32),
    )(u_rows, m_rows, wu, wm, b2)


_gather_seg = _make_gather(BSEG)


def kernel(users, movies, user_emb, movie_emb, fc_w, fc_b):
    wu = fc_w[:, :EMBED]
    wm = fc_w[:, EMBED:]
    b2 = fc_b.reshape(-1, 1)
    groups_seg = BSEG // CHUNK
    outs = []
    for s in range(NSEG):
        us = lax.slice(users, (s * BSEG,), ((s + 1) * BSEG,))
        ms = lax.slice(movies, (s * BSEG,), ((s + 1) * BSEG,))
        u_rows, m_rows = _gather_seg(us.reshape(groups_seg, CHUNK),
                                     ms.reshape(groups_seg, CHUNK),
                                     user_emb, movie_emb)
        outs.append(_matmul(u_rows, m_rows, wu, wm, b2))
    out_t = outs[0] if NSEG == 1 else jnp.concatenate(outs, axis=1)
    return out_t.T


# speedup vs baseline: 1.1502x; 1.0523x over previous
import functools

import jax
import jax.numpy as jnp
from jax import lax
from jax.experimental import pallas as pl
from jax.experimental.pallas import tpu as pltpu
from jax.experimental.pallas import tpu_sc as plsc

EMBED = 128
BATCH = 16384
CHUNK = 128
NC, NS = 2, 16
NW = NC * NS
NB = 3
NSEG = 1
BSEG = BATCH // NSEG

_sc_mesh = plsc.VectorSubcoreMesh(core_axis_name="c", subcore_axis_name="s")


def _gather_body(g_per_w, users_hbm, movies_hbm, uemb_hbm, memb_hbm,
                 u_out, m_out, idx_u, idx_m, rows, isem_u, isem_m, *sems):
    gsems, wsems = sems[:NB], sems[NB:]
    wid = lax.axis_index("s") * NC + lax.axis_index("c")
    gbase = wid * g_per_w
    icp_u = pltpu.async_copy(users_hbm.at[pl.ds(gbase, g_per_w)], idx_u, isem_u)
    icp_m = pltpu.async_copy(movies_hbm.at[pl.ds(gbase, g_per_w)], idx_m, isem_m)
    icp = {id(idx_u): icp_u, id(idx_m): icp_m}
    pairs = ([(idx_u, uemb_hbm, u_out, q) for q in range(g_per_w // 2)]
             + [(idx_m, memb_hbm, m_out, q) for q in range(g_per_w // 2)])
    n = len(pairs)
    gdesc, wdesc = [None] * n, [None] * n

    def issue_write(j):
        _, _, out, qj = pairs[j]
        bj = j % NB
        for d in gdesc[j]:
            d.wait()
        wdesc[j] = pltpu.async_copy(
            rows.at[bj],
            out.at[pl.ds((gbase // 2 + qj) * 2 * CHUNK, 2 * CHUNK)],
            wsems[bj])

    for c in range(n):
        b = c % NB
        if c >= NB:
            wdesc[c - NB].wait()
        ix, tab, _, q = pairs[c]
        d = icp.pop(id(ix), None)
        if d is not None:
            d.wait()
        gdesc[c] = [
            pltpu.async_copy(tab.at[ix.at[2 * q]],
                             rows.at[b, pl.ds(0, CHUNK)], gsems[b]),
            pltpu.async_copy(tab.at[ix.at[2 * q + 1]],
                             rows.at[b, pl.ds(CHUNK, CHUNK)], gsems[b]),
        ]
        if c - (NB - 1) >= 0:
            issue_write(c - (NB - 1))
    for j in range(max(n - (NB - 1), 0), n):
        issue_write(j)
    for j in range(max(n - NB, 0), n):
        wdesc[j].wait()


def _make_gather(batch):
    groups = batch // CHUNK
    g_per_w = groups // NW
    return pl.kernel(
        functools.partial(_gather_body, g_per_w),
        out_type=(
            jax.ShapeDtypeStruct((batch, EMBED), jnp.float32),
            jax.ShapeDtypeStruct((batch, EMBED), jnp.float32),
        ),
        mesh=_sc_mesh,
        scratch_types=(
            [pltpu.VMEM((g_per_w, CHUNK), jnp.int32),
             pltpu.VMEM((g_per_w, CHUNK), jnp.int32),
             pltpu.VMEM((NB, 2 * CHUNK, EMBED), jnp.float32),
             pltpu.SemaphoreType.DMA,
             pltpu.SemaphoreType.DMA]
            + [pltpu.SemaphoreType.DMA] * (2 * NB)
        ),
    )


def _mm_body(u_ref, m_ref, wu_ref, wm_ref, b_ref, o_ref):
    dn = (((1,), (1,)), ((), ()))
    acc = lax.dot_general(wu_ref[...], u_ref[...], dn,
                          preferred_element_type=jnp.float32)
    acc = acc + lax.dot_general(wm_ref[...], m_ref[...], dn,
                                preferred_element_type=jnp.float32)
    o_ref[...] = acc + b_ref[...]


BM = 4096


def _matmul(u_rows, m_rows, wu, wm, b2):
    batch = u_rows.shape[0]
    n_out = wu.shape[0]
    return pl.pallas_call(
        _mm_body,
        grid=(batch // BM,),
        in_specs=[
            pl.BlockSpec((BM, EMBED), lambda i: (i, 0)),
            pl.BlockSpec((BM, EMBED), lambda i: (i, 0)),
            pl.BlockSpec((n_out, EMBED), lambda i: (0, 0)),
            pl.BlockSpec((n_out, EMBED), lambda i: (0, 0)),
            pl.BlockSpec((n_out, 1), lambda i: (0, 0)),
        ],
        out_specs=pl.BlockSpec((n_out, BM), lambda i: (0, i)),
        out_shape=jax.ShapeDtypeStruct((n_out, batch), jnp.float32),
    )(u_rows, m_rows, wu, wm, b2)


_gather_seg = _make_gather(BSEG)


def kernel(users, movies, user_emb, movie_emb, fc_w, fc_b):
    wu = fc_w[:, :EMBED]
    wm = fc_w[:, EMBED:]
    b2 = fc_b.reshape(-1, 1)
    groups_seg = BSEG // CHUNK
    outs = []
    for s in range(NSEG):
        us = lax.slice(users, (s * BSEG,), ((s + 1) * BSEG,))
        ms = lax.slice(movies, (s * BSEG,), ((s + 1) * BSEG,))
        u_rows, m_rows = _gather_seg(us.reshape(groups_seg, CHUNK),
                                     ms.reshape(groups_seg, CHUNK),
                                     user_emb, movie_emb)
        outs.append(_matmul(u_rows, m_rows, wu, wm, b2))
    out_t = outs[0] if NSEG == 1 else jnp.concatenate(outs, axis=1)
    return out_t.T


# speedup vs baseline: 1.1510x; 1.0008x over previous
import functools

import jax
import jax.numpy as jnp
from jax import lax
from jax.experimental import pallas as pl
from jax.experimental.pallas import tpu as pltpu
from jax.experimental.pallas import tpu_sc as plsc

EMBED = 128
BATCH = 16384
CHUNK = 128
NC, NS = 2, 16
NW = NC * NS
NB = 3
NSEG = 1
BSEG = BATCH // NSEG

_sc_mesh = plsc.VectorSubcoreMesh(core_axis_name="c", subcore_axis_name="s")


def _gather_body(g_per_w, users_hbm, movies_hbm, uemb_hbm, memb_hbm,
                 u_out, m_out, idx_u, idx_m, rows, isem_u, isem_m, *sems):
    gsems, wsems = sems[:NB], sems[NB:]
    wid = lax.axis_index("s") * NC + lax.axis_index("c")
    gbase = wid * g_per_w
    icp_u = pltpu.async_copy(users_hbm.at[pl.ds(gbase, g_per_w)], idx_u, isem_u)
    icp_m = pltpu.async_copy(movies_hbm.at[pl.ds(gbase, g_per_w)], idx_m, isem_m)
    icp = {id(idx_u): icp_u, id(idx_m): icp_m}
    pairs = ([(idx_u, uemb_hbm, u_out, q) for q in range(g_per_w // 2)]
             + [(idx_m, memb_hbm, m_out, q) for q in range(g_per_w // 2)])
    n = len(pairs)
    gdesc, wdesc = [None] * n, [None] * n

    def issue_write(j):
        _, _, out, qj = pairs[j]
        bj = j % NB
        for d in gdesc[j]:
            d.wait()
        wdesc[j] = pltpu.async_copy(
            rows.at[bj],
            out.at[pl.ds((gbase // 2 + qj) * 2 * CHUNK, 2 * CHUNK)],
            wsems[bj])

    for c in range(n):
        b = c % NB
        if c >= NB:
            wdesc[c - NB].wait()
        ix, tab, _, q = pairs[c]
        d = icp.pop(id(ix), None)
        if d is not None:
            d.wait()
        gdesc[c] = [
            pltpu.async_copy(tab.at[ix.at[2 * q]],
                             rows.at[b, pl.ds(0, CHUNK)], gsems[b]),
            pltpu.async_copy(tab.at[ix.at[2 * q + 1]],
                             rows.at[b, pl.ds(CHUNK, CHUNK)], gsems[b]),
        ]
        if c - (NB - 1) >= 0:
            issue_write(c - (NB - 1))
    for j in range(max(n - (NB - 1), 0), n):
        issue_write(j)
    for j in range(max(n - NB, 0), n):
        wdesc[j].wait()


def _make_gather(batch):
    groups = batch // CHUNK
    g_per_w = groups // NW
    return pl.kernel(
        functools.partial(_gather_body, g_per_w),
        out_type=(
            jax.ShapeDtypeStruct((batch, EMBED), jnp.float32),
            jax.ShapeDtypeStruct((batch, EMBED), jnp.float32),
        ),
        mesh=_sc_mesh,
        scratch_types=(
            [pltpu.VMEM((g_per_w, CHUNK), jnp.int32),
             pltpu.VMEM((g_per_w, CHUNK), jnp.int32),
             pltpu.VMEM((NB, 2 * CHUNK, EMBED), jnp.float32),
             pltpu.SemaphoreType.DMA,
             pltpu.SemaphoreType.DMA]
            + [pltpu.SemaphoreType.DMA] * (2 * NB)
        ),
    )


def _mm_body(u_ref, m_ref, wu_ref, wm_ref, b_ref, o_ref):
    dn = (((1,), (1,)), ((), ()))
    acc = lax.dot_general(wu_ref[...], u_ref[...], dn,
                          preferred_element_type=jnp.float32)
    acc = acc + lax.dot_general(wm_ref[...], m_ref[...], dn,
                                preferred_element_type=jnp.float32)
    o_ref[...] = acc + b_ref[...]


BM = 8192


def _matmul(u_rows, m_rows, wu, wm, b2):
    batch = u_rows.shape[0]
    n_out = wu.shape[0]
    return pl.pallas_call(
        _mm_body,
        grid=(batch // BM,),
        in_specs=[
            pl.BlockSpec((BM, EMBED), lambda i: (i, 0)),
            pl.BlockSpec((BM, EMBED), lambda i: (i, 0)),
            pl.BlockSpec((n_out, EMBED), lambda i: (0, 0)),
            pl.BlockSpec((n_out, EMBED), lambda i: (0, 0)),
            pl.BlockSpec((n_out, 1), lambda i: (0, 0)),
        ],
        out_specs=pl.BlockSpec((n_out, BM), lambda i: (0, i)),
        out_shape=jax.ShapeDtypeStruct((n_out, batch), jnp.float32),
    )(u_rows, m_rows, wu, wm, b2)


_gather_seg = _make_gather(BSEG)


def kernel(users, movies, user_emb, movie_emb, fc_w, fc_b):
    wu = fc_w[:, :EMBED]
    wm = fc_w[:, EMBED:]
    b2 = fc_b.reshape(-1, 1)
    groups_seg = BSEG // CHUNK
    outs = []
    for s in range(NSEG):
        us = lax.slice(users, (s * BSEG,), ((s + 1) * BSEG,))
        ms = lax.slice(movies, (s * BSEG,), ((s + 1) * BSEG,))
        u_rows, m_rows = _gather_seg(us.reshape(groups_seg, CHUNK),
                                     ms.reshape(groups_seg, CHUNK),
                                     user_emb, movie_emb)
        outs.append(_matmul(u_rows, m_rows, wu, wm, b2))
    out_t = outs[0] if NSEG == 1 else jnp.concatenate(outs, axis=1)
    return out_t.T


# speedup vs baseline: 1.1520x; 1.0008x over previous
import functools

import jax
import jax.numpy as jnp
from jax import lax
from jax.experimental import pallas as pl
from jax.experimental.pallas import tpu as pltpu
from jax.experimental.pallas import tpu_sc as plsc

EMBED = 128
BATCH = 16384
CHUNK = 128
NC, NS = 2, 16
NW = NC * NS
NB = 3
NSEG = 1
BSEG = BATCH // NSEG

_sc_mesh = plsc.VectorSubcoreMesh(core_axis_name="c", subcore_axis_name="s")


def _gather_body(g_per_w, users_hbm, movies_hbm, uemb_hbm, memb_hbm,
                 u_out, m_out, idx_u, idx_m, rows, isem_u, isem_m, *sems):
    gsems, wsems = sems[:NB], sems[NB:]
    wid = lax.axis_index("s") * NC + lax.axis_index("c")
    gbase = wid * g_per_w
    icp_u = pltpu.async_copy(users_hbm.at[pl.ds(gbase, g_per_w)], idx_u, isem_u)
    icp_m = pltpu.async_copy(movies_hbm.at[pl.ds(gbase, g_per_w)], idx_m, isem_m)
    icp = {id(idx_u): icp_u, id(idx_m): icp_m}
    pairs = ([(idx_u, uemb_hbm, u_out, q) for q in range(g_per_w // 2)]
             + [(idx_m, memb_hbm, m_out, q) for q in range(g_per_w // 2)])
    n = len(pairs)
    gdesc, wdesc = [None] * n, [None] * n

    def issue_write(j):
        _, _, out, qj = pairs[j]
        bj = j % NB
        for d in gdesc[j]:
            d.wait()
        wdesc[j] = pltpu.async_copy(
            rows.at[bj],
            out.at[pl.ds((gbase // 2 + qj) * 2 * CHUNK, 2 * CHUNK)],
            wsems[bj])

    for c in range(n):
        b = c % NB
        if c >= NB:
            wdesc[c - NB].wait()
        ix, tab, _, q = pairs[c]
        d = icp.pop(id(ix), None)
        if d is not None:
            d.wait()
        gdesc[c] = [
            pltpu.async_copy(tab.at[ix.at[2 * q]],
                             rows.at[b, pl.ds(0, CHUNK)], gsems[b]),
            pltpu.async_copy(tab.at[ix.at[2 * q + 1]],
                             rows.at[b, pl.ds(CHUNK, CHUNK)], gsems[b]),
        ]
        if c - (NB - 1) >= 0:
            issue_write(c - (NB - 1))
    for j in range(max(n - (NB - 1), 0), n):
        issue_write(j)
    for j in range(max(n - NB, 0), n):
        wdesc[j].wait()


def _make_gather(batch):
    groups = batch // CHUNK
    g_per_w = groups // NW
    return pl.kernel(
        functools.partial(_gather_body, g_per_w),
        out_type=(
            jax.ShapeDtypeStruct((batch, EMBED), jnp.float32),
            jax.ShapeDtypeStruct((batch, EMBED), jnp.float32),
        ),
        mesh=_sc_mesh,
        scratch_types=(
            [pltpu.VMEM((g_per_w, CHUNK), jnp.int32),
             pltpu.VMEM((g_per_w, CHUNK), jnp.int32),
             pltpu.VMEM((NB, 2 * CHUNK, EMBED), jnp.float32),
             pltpu.SemaphoreType.DMA,
             pltpu.SemaphoreType.DMA]
            + [pltpu.SemaphoreType.DMA] * (2 * NB)
        ),
    )


def _mm_body(u_ref, m_ref, wu_ref, wm_ref, b_ref, o_ref):
    dn = (((1,), (1,)), ((), ()))
    acc = lax.dot_general(wu_ref[...], u_ref[...], dn,
                          preferred_element_type=jnp.float32)
    acc = acc + lax.dot_general(wm_ref[...], m_ref[...], dn,
                                preferred_element_type=jnp.float32)
    o_ref[...] = acc + b_ref[...]


BM = 4096


def _matmul(u_rows, m_rows, wu, wm, b2):
    batch = u_rows.shape[0]
    n_out = wu.shape[0]
    return pl.pallas_call(
        _mm_body,
        grid=(batch // BM,),
        in_specs=[
            pl.BlockSpec((BM, EMBED), lambda i: (i, 0)),
            pl.BlockSpec((BM, EMBED), lambda i: (i, 0)),
            pl.BlockSpec((n_out, EMBED), lambda i: (0, 0)),
            pl.BlockSpec((n_out, EMBED), lambda i: (0, 0)),
            pl.BlockSpec((n_out, 1), lambda i: (0, 0)),
        ],
        out_specs=pl.BlockSpec((n_out, BM), lambda i: (0, i)),
        out_shape=jax.ShapeDtypeStruct((n_out, batch), jnp.float32),
    )(u_rows, m_rows, wu, wm, b2)


_gather_seg = _make_gather(BSEG)


def kernel(users, movies, user_emb, movie_emb, fc_w, fc_b):
    wu = fc_w[:, :EMBED]
    wm = fc_w[:, EMBED:]
    b2 = fc_b.reshape(-1, 1)
    groups_seg = BSEG // CHUNK
    outs = []
    for s in range(NSEG):
        us = lax.slice(users, (s * BSEG,), ((s + 1) * BSEG,))
        ms = lax.slice(movies, (s * BSEG,), ((s + 1) * BSEG,))
        u_rows, m_rows = _gather_seg(us.reshape(groups_seg, CHUNK),
                                     ms.reshape(groups_seg, CHUNK),
                                     user_emb, movie_emb)
        outs.append(_matmul(u_rows, m_rows, wu, wm, b2))
    out_t = outs[0] if NSEG == 1 else jnp.concatenate(outs, axis=1)
    return out_t.T
